# Initial kernel scaffold; baseline (speedup 1.0000x reference)
import jax
import jax.numpy as jnp
from jax.experimental import pallas as pl


def kernel(node_attrs, batch, R, F_cut, electric_energy, atomic_electric_energy, short_energy, atomic_short_energy, ref_A, ref_B, ref_C, ref_D, ref_mu):
    raise NotImplementedError("write your pallas kernel here")


# speedup vs baseline: 1.8694x; 1.8694x over previous
import jax
import jax.numpy as jnp
from jax.experimental import pallas as pl
from jax.experimental.pallas import tpu as pltpu

N = 2048
TILE = 256
NT = N // TILE
NMOL = 16


def _energy_body(attrs_r_ref, types_c_ref, batch_rc_ref, batch_cl_ref, batch_rl_ref,
                 R_ref, F_ref, ee_ref, se_ref, aee_ref, ase_ref,
                 refA_ref, refB_ref, refC_ref, refD_ref, refmu_ref,
                 etot_ref, atomic_ref):
    r = pl.program_id(0)
    c = pl.program_id(1)

    refB = refB_ref[...]
    Ae = refA_ref[...] * jnp.exp(refB * refmu_ref[...])

    attrs_r = attrs_r_ref[...]
    rowsAe = jnp.dot(attrs_r, Ae, preferred_element_type=jnp.float32)
    rowsB = jnp.dot(attrs_r, refB, preferred_element_type=jnp.float32)
    rowsC = jnp.dot(attrs_r, refC_ref[...], preferred_element_type=jnp.float32)
    rowsD = jnp.dot(attrs_r, refD_ref[...], preferred_element_type=jnp.float32)

    tcol = types_c_ref[0]
    m0 = tcol == 0
    m2 = tcol == 2
    m3 = tcol == 3

    def pick(rows):
        x = jnp.where(m0, rows[:, 0:1], rows[:, 1:2])
        x = jnp.where(m2, rows[:, 2:3], x)
        x = jnp.where(m3, rows[:, 3:4], x)
        return x

    Aem = pick(rowsAe)
    Bm = pick(rowsB)
    Cm = pick(rowsC)
    Dm = pick(rowsD)

    R = R_ref[...]
    F = F_ref[...]
    r2 = R * R
    inv2 = 1.0 / r2
    inv4 = inv2 * inv2
    inv8 = inv4 * inv4
    e = Aem * jnp.exp(-Bm * R) - (Cm * r2 + Dm) * inv8

    rid = jax.lax.broadcasted_iota(jnp.int32, (TILE, 1), 0) + r * TILE
    cid = jax.lax.broadcasted_iota(jnp.int32, (1, TILE), 1) + c * TILE
    mask = (batch_rc_ref[...] == batch_cl_ref[0]) & (rid != cid)
    e = jnp.where(mask, e * F, 0.0)

    partial = jnp.sum(e, axis=1, keepdims=True) * 0.5

    @pl.when(c == 0)
    def _():
        atomic_ref[...] = aee_ref[...] + ase_ref[...] + partial

    @pl.when(c > 0)
    def _():
        atomic_ref[...] = atomic_ref[...] + partial

    batch_rl = batch_rl_ref[0]
    oh = (jax.lax.broadcasted_iota(jnp.int32, (NMOL, TILE), 0) == batch_rl)
    seg = jnp.dot(oh.astype(jnp.float32), partial, preferred_element_type=jnp.float32)

    first = (r == 0) & (c == 0)

    @pl.when(first)
    def _():
        etot_ref[...] = ee_ref[...] + se_ref[...] + seg

    @pl.when(jnp.logical_not(first))
    def _():
        etot_ref[...] = etot_ref[...] + seg


def kernel(node_attrs, batch, R, F_cut, electric_energy, atomic_electric_energy,
           short_energy, atomic_short_energy, ref_A, ref_B, ref_C, ref_D, ref_mu):
    types = jnp.argmax(node_attrs, axis=-1).astype(jnp.int32)
    types_3d = types.reshape(NT, 1, TILE)
    batch = batch.astype(jnp.int32)
    batch_rc = batch.reshape(N, 1)
    batch_3d = batch.reshape(NT, 1, TILE)

    grid = (NT, NT)
    small = pl.BlockSpec((4, 4), lambda r, c: (0, 0))
    in_specs = [
        pl.BlockSpec((TILE, 4), lambda r, c: (r, 0)),
        pl.BlockSpec((1, 1, TILE), lambda r, c: (c, 0, 0)),
        pl.BlockSpec((TILE, 1), lambda r, c: (r, 0)),
        pl.BlockSpec((1, 1, TILE), lambda r, c: (c, 0, 0)),
        pl.BlockSpec((1, 1, TILE), lambda r, c: (r, 0, 0)),
        pl.BlockSpec((TILE, TILE), lambda r, c: (r, c)),
        pl.BlockSpec((TILE, TILE), lambda r, c: (r, c)),
        pl.BlockSpec((NMOL, 1), lambda r, c: (0, 0)),
        pl.BlockSpec((NMOL, 1), lambda r, c: (0, 0)),
        pl.BlockSpec((TILE, 1), lambda r, c: (r, 0)),
        pl.BlockSpec((TILE, 1), lambda r, c: (r, 0)),
        small, small, small, small, small,
    ]
    out_specs = [
        pl.BlockSpec((NMOL, 1), lambda r, c: (0, 0)),
        pl.BlockSpec((TILE, 1), lambda r, c: (r, 0)),
    ]
    out_shape = [
        jax.ShapeDtypeStruct((NMOL, 1), jnp.float32),
        jax.ShapeDtypeStruct((N, 1), jnp.float32),
    ]
    etot, atomic = pl.pallas_call(
        _energy_body,
        grid=grid,
        in_specs=in_specs,
        out_specs=out_specs,
        out_shape=out_shape,
        compiler_params=pltpu.CompilerParams(
            dimension_semantics=("arbitrary", "arbitrary")),
    )(node_attrs, types_3d, batch_rc, batch_3d, batch_3d, R, F_cut,
      electric_energy, short_energy, atomic_electric_energy, atomic_short_energy,
      ref_A, ref_B, ref_C, ref_D, ref_mu)
    return (etot, atomic)


# speedup vs baseline: 2.0209x; 1.0810x over previous
import jax
import jax.numpy as jnp
from jax.experimental import pallas as pl
from jax.experimental.pallas import tpu as pltpu

N = 2048
TILE = 256
NT = N // TILE
NMOL = 16


def _energy_body(cmap_ref, nact_ref,
                 attrs_r_ref, types_c_ref, batch_rc_ref, batch_cl_ref, batch_rl_ref,
                 R_ref, F_ref, ee_ref, se_ref, aee_ref, ase_ref,
                 refA_ref, refB_ref, refC_ref, refD_ref, refmu_ref,
                 etot_ref, atomic_ref):
    r = pl.program_id(0)
    c = pl.program_id(1)
    active = c < nact_ref[r]

    @pl.when(active)
    def _do():
        _tile_compute(cmap_ref, attrs_r_ref, types_c_ref, batch_rc_ref,
                      batch_cl_ref, batch_rl_ref, R_ref, F_ref, ee_ref, se_ref,
                      aee_ref, ase_ref, refA_ref, refB_ref, refC_ref, refD_ref,
                      refmu_ref, etot_ref, atomic_ref, r, c)


def _tile_compute(cmap_ref, attrs_r_ref, types_c_ref, batch_rc_ref, batch_cl_ref,
                  batch_rl_ref, R_ref, F_ref, ee_ref, se_ref, aee_ref, ase_ref,
                  refA_ref, refB_ref, refC_ref, refD_ref, refmu_ref,
                  etot_ref, atomic_ref, r, c):
    cb = cmap_ref[r, c]

    refB = refB_ref[...]
    Ae = refA_ref[...] * jnp.exp(refB * refmu_ref[...])

    attrs_r = attrs_r_ref[...]
    rowsAe = jnp.dot(attrs_r, Ae, preferred_element_type=jnp.float32)
    rowsB = jnp.dot(attrs_r, refB, preferred_element_type=jnp.float32)
    rowsC = jnp.dot(attrs_r, refC_ref[...], preferred_element_type=jnp.float32)
    rowsD = jnp.dot(attrs_r, refD_ref[...], preferred_element_type=jnp.float32)

    tcol = types_c_ref[0]
    m0 = tcol == 0
    m2 = tcol == 2
    m3 = tcol == 3

    def pick(rows):
        x = jnp.where(m0, rows[:, 0:1], rows[:, 1:2])
        x = jnp.where(m2, rows[:, 2:3], x)
        x = jnp.where(m3, rows[:, 3:4], x)
        return x

    Aem = pick(rowsAe)
    Bm = pick(rowsB)
    Cm = pick(rowsC)
    Dm = pick(rowsD)

    R = R_ref[...]
    F = F_ref[...]
    r2 = R * R
    inv2 = 1.0 / r2
    inv4 = inv2 * inv2
    inv8 = inv4 * inv4
    e = Aem * jnp.exp(-Bm * R) - (Cm * r2 + Dm) * inv8

    rid = jax.lax.broadcasted_iota(jnp.int32, (TILE, 1), 0) + r * TILE
    cid = jax.lax.broadcasted_iota(jnp.int32, (1, TILE), 1) + cb * TILE
    mask = (batch_rc_ref[...] == batch_cl_ref[0]) & (rid != cid)
    e = jnp.where(mask, e * F, 0.0)

    partial = jnp.sum(e, axis=1, keepdims=True) * 0.5

    @pl.when(c == 0)
    def _():
        atomic_ref[...] = aee_ref[...] + ase_ref[...] + partial

    @pl.when(c > 0)
    def _():
        atomic_ref[...] = atomic_ref[...] + partial

    batch_rl = batch_rl_ref[0]
    oh = (jax.lax.broadcasted_iota(jnp.int32, (NMOL, TILE), 0) == batch_rl)
    seg = jnp.dot(oh.astype(jnp.float32), partial, preferred_element_type=jnp.float32)

    first = (r == 0) & (c == 0)

    @pl.when(first)
    def _():
        etot_ref[...] = ee_ref[...] + se_ref[...] + seg

    @pl.when(jnp.logical_not(first))
    def _():
        etot_ref[...] = etot_ref[...] + seg


def kernel(node_attrs, batch, R, F_cut, electric_energy, atomic_electric_energy,
           short_energy, atomic_short_energy, ref_A, ref_B, ref_C, ref_D, ref_mu):
    types = jnp.argmax(node_attrs, axis=-1).astype(jnp.int32)
    types_3d = types.reshape(NT, 1, TILE)
    batch = batch.astype(jnp.int32)
    batch_rc = batch.reshape(N, 1)
    batch_3d = batch.reshape(NT, 1, TILE)

    m_lo = batch[::TILE]
    m_hi = batch[TILE - 1::TILE]
    col_lo = jnp.searchsorted(batch, m_lo, side='left')
    col_hi = jnp.searchsorted(batch, m_hi, side='right')
    lo_b = (col_lo // TILE).astype(jnp.int32)
    hi_b = ((col_hi - 1) // TILE).astype(jnp.int32)
    nact = hi_b - lo_b + 1
    cmap = jnp.minimum(lo_b[:, None] + jnp.arange(NT, dtype=jnp.int32)[None, :],
                       hi_b[:, None])

    small = pl.BlockSpec((4, 4), lambda r, c, cm, na: (0, 0))
    in_specs = [
        pl.BlockSpec((TILE, 4), lambda r, c, cm, na: (r, 0)),
        pl.BlockSpec((1, 1, TILE), lambda r, c, cm, na: (cm[r, c], 0, 0)),
        pl.BlockSpec((TILE, 1), lambda r, c, cm, na: (r, 0)),
        pl.BlockSpec((1, 1, TILE), lambda r, c, cm, na: (cm[r, c], 0, 0)),
        pl.BlockSpec((1, 1, TILE), lambda r, c, cm, na: (r, 0, 0)),
        pl.BlockSpec((TILE, TILE), lambda r, c, cm, na: (r, cm[r, c])),
        pl.BlockSpec((TILE, TILE), lambda r, c, cm, na: (r, cm[r, c])),
        pl.BlockSpec((NMOL, 1), lambda r, c, cm, na: (0, 0)),
        pl.BlockSpec((NMOL, 1), lambda r, c, cm, na: (0, 0)),
        pl.BlockSpec((TILE, 1), lambda r, c, cm, na: (r, 0)),
        pl.BlockSpec((TILE, 1), lambda r, c, cm, na: (r, 0)),
        small, small, small, small, small,
    ]
    out_specs = [
        pl.BlockSpec((NMOL, 1), lambda r, c, cm, na: (0, 0)),
        pl.BlockSpec((TILE, 1), lambda r, c, cm, na: (r, 0)),
    ]
    out_shape = [
        jax.ShapeDtypeStruct((NMOL, 1), jnp.float32),
        jax.ShapeDtypeStruct((N, 1), jnp.float32),
    ]
    grid_spec = pltpu.PrefetchScalarGridSpec(
        num_scalar_prefetch=2,
        grid=(NT, NT),
        in_specs=in_specs,
        out_specs=out_specs,
    )
    etot, atomic = pl.pallas_call(
        _energy_body,
        grid_spec=grid_spec,
        out_shape=out_shape,
        compiler_params=pltpu.CompilerParams(
            dimension_semantics=("arbitrary", "arbitrary")),
    )(cmap, nact, node_attrs, types_3d, batch_rc, batch_3d, batch_3d, R, F_cut,
      electric_energy, short_energy, atomic_electric_energy, atomic_short_energy,
      ref_A, ref_B, ref_C, ref_D, ref_mu)
    return (etot, atomic)
